# Initial kernel scaffold; baseline (speedup 1.0000x reference)
import jax
import jax.numpy as jnp
from jax.experimental import pallas as pl


def kernel(x, edge_index, epsilon, W1, b1, g1, beta1, W2, b2, g2, beta2):
    raise NotImplementedError("write your pallas kernel here")


# speedup vs baseline: 6.4245x; 6.4245x over previous
import jax
import jax.numpy as jnp
from jax import lax
from jax.experimental import pallas as pl
from jax.experimental.pallas import tpu as pltpu
from jax.experimental.pallas import tpu_sc as plsc

N = 10000
NPAD = 10240
E = 320000
F = 128
NC = 2
NS = 16
NW = NC * NS
EPW = E // NW
CHUNK = 80
NCHUNK = EPW // CHUNK
RPT = NPAD // NS
RBLK = 80
NRB = RPT // RBLK


def _sc_agg_body(x_hbm, src_hbm, dst_hbm, zero_hbm, out_hbm,
                 src_v, dst_v, rows_v, acc_sh, sem):
    cid = lax.axis_index("c")
    sid = lax.axis_index("s")
    w = cid * NS + sid

    pltpu.sync_copy(src_hbm.at[w], src_v)
    pltpu.sync_copy(dst_hbm.at[w], dst_v)

    pltpu.sync_copy(zero_hbm, rows_v)
    for c in range(NRB):
        pltpu.sync_copy(rows_v, acc_sh.at[pl.ds(sid * RPT + c * RBLK, RBLK)])
    plsc.subcore_barrier()

    def body(j, carry):
        pltpu.async_copy(x_hbm.at[src_v.at[j]], rows_v, sem).wait()
        pltpu.sync_copy(rows_v, acc_sh.at[dst_v.at[j]], add=True)
        return carry

    lax.fori_loop(0, NCHUNK, body, 0)
    plsc.subcore_barrier()

    for c in range(NRB):
        r0 = sid * RPT + c * RBLK
        pltpu.sync_copy(acc_sh.at[pl.ds(r0, RBLK)], rows_v)
        pltpu.sync_copy(rows_v, out_hbm.at[cid, pl.ds(r0, RBLK)])


def _sc_aggregate(x, src3, dst3, zeros_blk):
    return pl.kernel(
        _sc_agg_body,
        out_type=jax.ShapeDtypeStruct((NC, NPAD, F), jnp.float32),
        mesh=plsc.VectorSubcoreMesh(core_axis_name="c", subcore_axis_name="s",
                                    num_cores=NC, num_subcores=NS),
        scratch_types=[
            pltpu.VMEM((NCHUNK, CHUNK), jnp.int32),
            pltpu.VMEM((NCHUNK, CHUNK), jnp.int32),
            pltpu.VMEM((CHUNK, F), jnp.float32),
            pltpu.VMEM_SHARED((NPAD, F), jnp.float32),
            pltpu.SemaphoreType.DMA,
        ],
    )(x, src3, dst3, zeros_blk)


def _bn_elu(h, g, beta):
    mu = jnp.mean(h, axis=0, keepdims=True)
    d = h - mu
    var = jnp.mean(d * d, axis=0, keepdims=True)
    hn = d * lax.rsqrt(var + 1e-5) * g + beta
    return jnp.where(hn > 0, hn, jnp.exp(jnp.minimum(hn, 0.0)) - 1.0)


def _tc_mlp_body(part_ref, x_ref, eps_ref, w1t_ref, b1_ref, g1_ref, bt1_ref,
                 w2t_ref, b2_ref, g2_ref, bt2_ref, out_ref):
    agg = (part_ref[0, :N, :] + part_ref[1, :N, :]
           + eps_ref[0, 0] * x_ref[...])
    h = jnp.dot(agg, w1t_ref[...], precision=lax.Precision.HIGHEST)
    h = _bn_elu(h + b1_ref[...], g1_ref[...], bt1_ref[...])
    h = jnp.dot(h, w2t_ref[...], precision=lax.Precision.HIGHEST)
    out_ref[...] = _bn_elu(h + b2_ref[...], g2_ref[...], bt2_ref[...])


_tc_mlp = pl.pallas_call(
    _tc_mlp_body,
    out_shape=jax.ShapeDtypeStruct((N, F), jnp.float32),
)


def kernel(x, edge_index, epsilon, W1, b1, g1, beta1, W2, b2, g2, beta2):
    dst3 = edge_index[0].reshape(NW, NCHUNK, CHUNK)
    src3 = edge_index[1].reshape(NW, NCHUNK, CHUNK)
    zeros_blk = jnp.zeros((RBLK, F), jnp.float32)
    part = _sc_aggregate(x, src3, dst3, zeros_blk)
    return _tc_mlp(part, x, epsilon,
                   W1.T, b1.reshape(1, F), g1.reshape(1, F),
                   beta1.reshape(1, F),
                   W2.T, b2.reshape(1, F), g2.reshape(1, F),
                   beta2.reshape(1, F))


# speedup vs baseline: 9.6402x; 1.5005x over previous
import jax
import jax.numpy as jnp
from jax import lax
from jax.experimental import pallas as pl
from jax.experimental.pallas import tpu as pltpu
from jax.experimental.pallas import tpu_sc as plsc

N = 10000
NPAD = 10240
E = 320000
F = 128
NC = 2
NS = 16
NW = NC * NS
EPW = E // NW
CHUNK = 80
NCHUNK = EPW // CHUNK
RPT = NPAD // NS
RBLK = 80
NRB = RPT // RBLK
L = 16


def _sc_agg_body(x_hbm, pk_hbm, zero_hbm, out_hbm,
                 pk_v, gidx, sidx, rows2, acc_sh, sem):
    cid = lax.axis_index("c")
    sid = lax.axis_index("s")
    w = cid * NS + sid

    pltpu.sync_copy(pk_hbm.at[w], pk_v)

    pltpu.sync_copy(zero_hbm, rows2.at[0])
    for c in range(NRB):
        pltpu.sync_copy(rows2.at[0],
                        acc_sh.at[pl.ds(sid * RPT + c * RBLK, RBLK)])
    plsc.subcore_barrier()

    def unpack(j, slot):
        for c in range(CHUNK // L):
            v = pk_v[j, pl.ds(c * L, L)]
            gidx[slot, pl.ds(c * L, L)] = v & 0xFFFF
            sidx[slot, pl.ds(c * L, L)] = v >> 16

    def issue_gather(slot):
        pltpu.async_copy(x_hbm.at[gidx.at[slot]], rows2.at[slot], sem)

    unpack(0, 0)
    issue_gather(0)
    unpack(1, 1)

    def body(j, carry):
        b = lax.rem(j, 2)

        @pl.when(j + 1 < NCHUNK)
        def _():
            issue_gather(1 - b)

        pltpu.make_async_copy(x_hbm.at[gidx.at[b]], rows2.at[b], sem).wait()
        pltpu.sync_copy(rows2.at[b], acc_sh.at[sidx.at[b]], add=True)

        @pl.when(j + 2 < NCHUNK)
        def _():
            unpack(j + 2, b)

        return carry

    lax.fori_loop(0, NCHUNK, body, 0)
    plsc.subcore_barrier()

    for c in range(NRB):
        r0 = sid * RPT + c * RBLK
        pltpu.sync_copy(acc_sh.at[pl.ds(r0, RBLK)], rows2.at[0])
        pltpu.sync_copy(rows2.at[0], out_hbm.at[cid, pl.ds(r0, RBLK)])


def _sc_aggregate(x, pk3, zeros_blk):
    return pl.kernel(
        _sc_agg_body,
        out_type=jax.ShapeDtypeStruct((NC, NPAD, F), jnp.float32),
        mesh=plsc.VectorSubcoreMesh(core_axis_name="c", subcore_axis_name="s",
                                    num_cores=NC, num_subcores=NS),
        scratch_types=[
            pltpu.VMEM((NCHUNK, CHUNK), jnp.int32),
            pltpu.VMEM((2, CHUNK), jnp.int32),
            pltpu.VMEM((2, CHUNK), jnp.int32),
            pltpu.VMEM((2, CHUNK, F), jnp.float32),
            pltpu.VMEM_SHARED((NPAD, F), jnp.float32),
            pltpu.SemaphoreType.DMA,
        ],
    )(x, pk3, zeros_blk)


def _bn_elu(h, g, beta):
    mu = jnp.mean(h, axis=0, keepdims=True)
    d = h - mu
    var = jnp.mean(d * d, axis=0, keepdims=True)
    hn = d * lax.rsqrt(var + 1e-5) * g + beta
    return jnp.where(hn > 0, hn, jnp.exp(jnp.minimum(hn, 0.0)) - 1.0)


def _tc_mlp_body(part_ref, x_ref, eps_ref, w1t_ref, b1_ref, g1_ref, bt1_ref,
                 w2t_ref, b2_ref, g2_ref, bt2_ref, out_ref):
    agg = (part_ref[0, :N, :] + part_ref[1, :N, :]
           + eps_ref[0, 0] * x_ref[...])
    h = jnp.dot(agg, w1t_ref[...], precision=lax.Precision.HIGHEST)
    h = _bn_elu(h + b1_ref[...], g1_ref[...], bt1_ref[...])
    h = jnp.dot(h, w2t_ref[...], precision=lax.Precision.HIGHEST)
    out_ref[...] = _bn_elu(h + b2_ref[...], g2_ref[...], bt2_ref[...])


_tc_mlp = pl.pallas_call(
    _tc_mlp_body,
    out_shape=jax.ShapeDtypeStruct((N, F), jnp.float32),
)


def kernel(x, edge_index, epsilon, W1, b1, g1, beta1, W2, b2, g2, beta2):
    packed = (edge_index[0] << 16) | edge_index[1]
    pk3 = packed.reshape(NW, NCHUNK, CHUNK)
    zeros_blk = jnp.zeros((RBLK, F), jnp.float32)
    part = _sc_aggregate(x, pk3, zeros_blk)
    return _tc_mlp(part, x, epsilon,
                   W1.T, b1.reshape(1, F), g1.reshape(1, F),
                   beta1.reshape(1, F),
                   W2.T, b2.reshape(1, F), g2.reshape(1, F),
                   beta2.reshape(1, F))
